# baseline (speedup 1.0000x reference)
import functools

import jax
import jax.numpy as jnp
from jax import lax
from jax.experimental import pallas as pl
from jax.experimental.pallas import tpu as pltpu
from jax.experimental.pallas import tpu_sc as plsc

N = 1600000
EPS = 1e-08
MMR, MMC = 1250, 1280

B = 6400
C = B // 16
NW1 = N // B
FB = 512
FR = FB // 2
FC = FB // 16
NW2 = NW1 - 1
NWORK = 32
WPT = -(-NW1 // NWORK)

_mesh = plsc.VectorSubcoreMesh(
    core_axis_name="c", subcore_axis_name="s", num_cores=2, num_subcores=16
)
_params = pltpu.CompilerParams(needs_layout_passes=False)


def _mm_body(s_ref, o_ref):
    s = s_ref[...]
    o_ref[0:1, :] = jnp.full((1, 128), jnp.min(s), jnp.float32)
    o_ref[1:2, :] = jnp.full((1, 128), jnp.max(s), jnp.float32)


def _zero_hist(hist):
    def body(t, c):
        hist[pl.ds(t * 16, 16)] = jnp.zeros((16,), jnp.int32)
        return c

    lax.fori_loop(0, 16, body, None)


def _scan_hist2(h0, h1):
    def body(t, carry):
        sl = pl.ds(t * 16, 16)
        l = h0[sl]
        r = h1[sl]
        tot = l + r
        inc = plsc.cumsum(tot)
        e = inc - tot + carry
        h0[sl] = e
        h1[sl] = e + l
        return carry + jnp.sum(tot)

    lax.fori_loop(0, 16, body, jnp.int32(0))


def _perm_pass(h0, h1, hn0, hn1, kin, pin, vin, kout, pout, vout,
               shift, shift_nxt, base, nch):
    nh = nch // 2
    he = nh * 16
    ones = jnp.ones((16,), jnp.int32)

    def perm_body(j, c):
        for u in range(2):
            for s, hist in ((0, h0), (1, h1)):
                sl = pl.ds((s * nh + j * 2 + u) * 16, 16)
                k = kin[sl]
                p = pin[sl]
                v = vin[sl]
                d = lax.shift_right_logical(k - base, shift) & 255
                cnt, last = plsc.scan_count(d)
                off = plsc.load_gather(hist, [d]) + cnt - 1
                plsc.store_scatter(kout, [off], k)
                plsc.store_scatter(pout, [off], p)
                plsc.store_scatter(vout, [off], v)
                plsc.addupdate_scatter(hist, [d], cnt, mask=last)
                if shift_nxt is not None:
                    d2 = lax.shift_right_logical(k - base, shift_nxt) & 255
                    plsc.addupdate_scatter(hn0, [d2], ones, mask=off < he)
                    plsc.addupdate_scatter(hn1, [d2], ones, mask=off >= he)
        return c

    lax.fori_loop(0, nh // 2, perm_body, None)


def _sort_window(ha0, ha1, hb0, hb1, ka, pa, va, kb, pb, vb, kmin, kmax, nch):
    base = kmin & jnp.int32(-256)
    span = kmax - base
    _scan_hist2(ha0, ha1)
    _zero_hist(hb0)
    _zero_hist(hb1)
    _perm_pass(ha0, ha1, hb0, hb1, ka, pa, va, kb, pb, vb, 0, 8, base, nch)
    _scan_hist2(hb0, hb1)
    _zero_hist(ha0)
    _zero_hist(ha1)
    _perm_pass(hb0, hb1, ha0, ha1, kb, pb, vb, ka, pa, va, 8, 16, base, nch)

    @pl.when(span >= (1 << 16))
    def _():
        _scan_hist2(ha0, ha1)
        _zero_hist(hb0)
        _zero_hist(hb1)
        _perm_pass(ha0, ha1, hb0, hb1, ka, pa, va, kb, pb, vb, 16, 24, base, nch)

        @pl.when(span >= (1 << 24))
        def _():
            _scan_hist2(hb0, hb1)
            _perm_pass(hb0, hb1, hb0, hb1, kb, pb, vb, ka, pa, va,
                       24, None, base, nch)

    return (span >= (1 << 16)) & (span < (1 << 24))


def _minmax_hist2(ref, h0, h1, nch):
    nh = nch // 2
    ones = jnp.ones((16,), jnp.int32)

    def body(j, carry):
        for s, h in ((0, h0), (1, h1)):
            k = ref[pl.ds((s * nh + j) * 16, 16)]
            plsc.addupdate_scatter(h, [k & 255], ones)
            carry = (jnp.minimum(carry[0], k), jnp.maximum(carry[1], k))
        return carry

    init = (jnp.full((16,), jnp.int32(2**31 - 1)),
            jnp.full((16,), jnp.int32(-(2**31))))
    kmin_v, kmax_v = lax.fori_loop(0, nh, body, init)
    return jnp.min(kmin_v), jnp.max(kmax_v)


def _phase1_body(src_hbm, idx_hbm, mm_hbm, ko_hbm, po_hbm, vo_hbm,
                 mmv, seg, ka, pa, va, kb, pb, vb, ha0, ha1, hb0, hb1):
    wid = lax.axis_index("s") * 2 + lax.axis_index("c")
    lane = lax.iota(jnp.int32, 16)
    pltpu.sync_copy(mm_hbm, mmv)
    mn_vec = mmv[pl.ds(0, 16)]
    inv_vec = 1.0 / (mmv[pl.ds(128, 16)] - mn_vec + jnp.float32(EPS))

    def tile_body(i, _):
        w = i * NWORK + wid

        @pl.when(w < NW1)
        def _():
            base = w * B
            pltpu.sync_copy(src_hbm.at[pl.ds(base, B)], va)
            pltpu.sync_copy(idx_hbm.at[pl.ds(base, B)], seg)
            _zero_hist(ha0)
            _zero_hist(ha1)
            ones = jnp.ones((16,), jnp.int32)

            def key_body(j, carry):
                for s, h in ((0, ha0), (1, ha1)):
                    jj = s * (C // 2) + j
                    sl = pl.ds(jj * 16, 16)
                    nrm = ((va[sl] - mn_vec) * inv_vec
                           + seg[sl].astype(jnp.float32))
                    kbits = plsc.bitcast(nrm, jnp.int32)
                    ka[sl] = kbits
                    pa[sl] = base + jj * 16 + lane
                    plsc.addupdate_scatter(h, [kbits & 255], ones)
                    carry = (jnp.minimum(carry[0], kbits),
                             jnp.maximum(carry[1], kbits))
                return carry

            init = (jnp.full((16,), jnp.int32(2**31 - 1)),
                    jnp.full((16,), jnp.int32(-(2**31))))
            kmin_v, kmax_v = lax.fori_loop(0, C // 2, key_body, init)
            in_b = _sort_window(ha0, ha1, hb0, hb1, ka, pa, va, kb, pb, vb,
                                jnp.min(kmin_v), jnp.max(kmax_v), C)

            @pl.when(in_b)
            def _():
                pltpu.sync_copy(kb.at[pl.ds(0, FR)], ko_hbm.at[pl.ds(base, FR)])
                pltpu.sync_copy(kb.at[pl.ds(B - FR, FR)],
                                ko_hbm.at[pl.ds(base + B - FR, FR)])
                pltpu.sync_copy(pb, po_hbm.at[pl.ds(base, B)])
                pltpu.sync_copy(vb, vo_hbm.at[pl.ds(base, B)])

            @pl.when(jnp.logical_not(in_b))
            def _():
                pltpu.sync_copy(ka.at[pl.ds(0, FR)], ko_hbm.at[pl.ds(base, FR)])
                pltpu.sync_copy(ka.at[pl.ds(B - FR, FR)],
                                ko_hbm.at[pl.ds(base + B - FR, FR)])
                pltpu.sync_copy(pa, po_hbm.at[pl.ds(base, B)])
                pltpu.sync_copy(va, vo_hbm.at[pl.ds(base, B)])

        return _

    lax.fori_loop(0, WPT, tile_body, None)


def _phase2_body(ki_hbm, pi_hbm, vi_hbm, srt_hbm, perm_hbm,
                 ka, pa, va, kb, pb, vb, cp, cv, ha0, ha1, hb0, hb1):
    wid = lax.axis_index("s") * 2 + lax.axis_index("c")

    def copy_body(i, _):
        w = i * NWORK + wid

        @pl.when(w == 0)
        def _():
            pltpu.sync_copy(pi_hbm.at[pl.ds(0, B - FR)], cp.at[pl.ds(0, B - FR)])
            pltpu.sync_copy(vi_hbm.at[pl.ds(0, B - FR)], cv.at[pl.ds(0, B - FR)])
            pltpu.sync_copy(cp.at[pl.ds(0, B - FR)], perm_hbm.at[pl.ds(0, B - FR)])
            pltpu.sync_copy(cv.at[pl.ds(0, B - FR)], srt_hbm.at[pl.ds(0, B - FR)])

        @pl.when((w > 0) & (w < NW1 - 1))
        def _():
            s = w * B + FR
            pltpu.sync_copy(pi_hbm.at[pl.ds(s, B - FB)], cp.at[pl.ds(0, B - FB)])
            pltpu.sync_copy(vi_hbm.at[pl.ds(s, B - FB)], cv.at[pl.ds(0, B - FB)])
            pltpu.sync_copy(cp.at[pl.ds(0, B - FB)], perm_hbm.at[pl.ds(s, B - FB)])
            pltpu.sync_copy(cv.at[pl.ds(0, B - FB)], srt_hbm.at[pl.ds(s, B - FB)])

        @pl.when(w == NW1 - 1)
        def _():
            s = (NW1 - 1) * B + FR
            pltpu.sync_copy(pi_hbm.at[pl.ds(s, B - FR)], cp.at[pl.ds(0, B - FR)])
            pltpu.sync_copy(vi_hbm.at[pl.ds(s, B - FR)], cv.at[pl.ds(0, B - FR)])
            pltpu.sync_copy(cp.at[pl.ds(0, B - FR)], perm_hbm.at[pl.ds(s, B - FR)])
            pltpu.sync_copy(cv.at[pl.ds(0, B - FR)], srt_hbm.at[pl.ds(s, B - FR)])

        return _

    lax.fori_loop(0, WPT, copy_body, None)

    def fix_body(i, _):
        w = i * NWORK + wid

        @pl.when(w < NW2)
        def _():
            base = (w + 1) * B - FR
            pltpu.sync_copy(ki_hbm.at[pl.ds(base, FB)], ka)
            pltpu.sync_copy(pi_hbm.at[pl.ds(base, FB)], pa)
            pltpu.sync_copy(vi_hbm.at[pl.ds(base, FB)], va)
            _zero_hist(ha0)
            _zero_hist(ha1)
            kmin, kmax = _minmax_hist2(ka, ha0, ha1, FC)
            in_b = _sort_window(ha0, ha1, hb0, hb1, ka, pa, va, kb, pb, vb,
                                kmin, kmax, FC)

            @pl.when(in_b)
            def _():
                pltpu.sync_copy(pb, perm_hbm.at[pl.ds(base, FB)])
                pltpu.sync_copy(vb, srt_hbm.at[pl.ds(base, FB)])

            @pl.when(jnp.logical_not(in_b))
            def _():
                pltpu.sync_copy(pa, perm_hbm.at[pl.ds(base, FB)])
                pltpu.sync_copy(va, srt_hbm.at[pl.ds(base, FB)])

        return _

    lax.fori_loop(0, WPT, fix_body, None)


_phase1 = functools.partial(
    pl.kernel,
    out_type=[
        jax.ShapeDtypeStruct((N,), jnp.int32),
        jax.ShapeDtypeStruct((N,), jnp.int32),
        jax.ShapeDtypeStruct((N,), jnp.float32),
    ],
    mesh=_mesh,
    compiler_params=_params,
    scratch_types=[
        pltpu.VMEM((256,), jnp.float32),
        pltpu.VMEM((B,), jnp.int32),
        pltpu.VMEM((B,), jnp.int32),
        pltpu.VMEM((B,), jnp.int32),
        pltpu.VMEM((B,), jnp.float32),
        pltpu.VMEM((B,), jnp.int32),
        pltpu.VMEM((B,), jnp.int32),
        pltpu.VMEM((B,), jnp.float32),
        pltpu.VMEM((256,), jnp.int32),
        pltpu.VMEM((256,), jnp.int32),
        pltpu.VMEM((256,), jnp.int32),
        pltpu.VMEM((256,), jnp.int32),
    ],
)(_phase1_body)

_phase2 = functools.partial(
    pl.kernel,
    out_type=[
        jax.ShapeDtypeStruct((N,), jnp.float32),
        jax.ShapeDtypeStruct((N,), jnp.int32),
    ],
    mesh=_mesh,
    compiler_params=_params,
    scratch_types=[
        pltpu.VMEM((FB,), jnp.int32),
        pltpu.VMEM((FB,), jnp.int32),
        pltpu.VMEM((FB,), jnp.float32),
        pltpu.VMEM((FB,), jnp.int32),
        pltpu.VMEM((FB,), jnp.int32),
        pltpu.VMEM((FB,), jnp.float32),
        pltpu.VMEM((B,), jnp.int32),
        pltpu.VMEM((B,), jnp.float32),
        pltpu.VMEM((256,), jnp.int32),
        pltpu.VMEM((256,), jnp.int32),
        pltpu.VMEM((256,), jnp.int32),
        pltpu.VMEM((256,), jnp.int32),
    ],
)(_phase2_body)


def kernel(src, index):
    mm = pl.pallas_call(
        _mm_body,
        out_shape=jax.ShapeDtypeStruct((2, 128), jnp.float32),
    )(src.reshape(MMR, MMC))
    key1, pos1, val1 = _phase1(src, index.astype(jnp.int32), mm.reshape(256))
    srt, perm = _phase2(key1, pos1, val1)
    return srt, perm


# speedup vs baseline: 1.0066x; 1.0066x over previous
import functools

import jax
import jax.numpy as jnp
from jax import lax
from jax.experimental import pallas as pl
from jax.experimental.pallas import tpu as pltpu
from jax.experimental.pallas import tpu_sc as plsc

N = 1600000
EPS = 1e-08
MMR, MMC = 1250, 1280

B = 6400
C = B // 16
NW1 = N // B
FB = 512
FR = FB // 2
FC = FB // 16
NW2 = NW1 - 1
NWORK = 32
WPT = -(-NW1 // NWORK)

_mesh = plsc.VectorSubcoreMesh(
    core_axis_name="c", subcore_axis_name="s", num_cores=2, num_subcores=16
)
_params = pltpu.CompilerParams(needs_layout_passes=False)


def _mm_body(s_ref, o_ref):
    s = s_ref[...]
    o_ref[0:1, :] = jnp.full((1, 128), jnp.min(s), jnp.float32)
    o_ref[1:2, :] = jnp.full((1, 128), jnp.max(s), jnp.float32)


def _zero_hist(hist):
    def body(t, c):
        hist[pl.ds(t * 16, 16)] = jnp.zeros((16,), jnp.int32)
        return c

    lax.fori_loop(0, 16, body, None)


def _scan_hist(hist):
    def body(t, carry):
        sl = pl.ds(t * 16, 16)
        h = hist[sl]
        inc = plsc.cumsum(h)
        hist[sl] = inc - h + carry
        return carry + jnp.sum(h)

    lax.fori_loop(0, 16, body, jnp.int32(0))


def _perm_pass(hist, hnxt, kin, pin, vin, kout, pout, vout,
               shift, shift_nxt, base, nch):
    def perm_body(j, c):
        for u in range(8):
            sl = pl.ds((j * 8 + u) * 16, 16)
            k = kin[sl]
            p = pin[sl]
            v = vin[sl]
            d = lax.shift_right_logical(k - base, shift) & 255
            cnt, last = plsc.scan_count(d)
            off = plsc.load_gather(hist, [d]) + cnt - 1
            plsc.store_scatter(kout, [off], k)
            plsc.store_scatter(pout, [off], p)
            plsc.store_scatter(vout, [off], v)
            plsc.addupdate_scatter(hist, [d], cnt, mask=last)
            if shift_nxt is not None:
                d2 = lax.shift_right_logical(k - base, shift_nxt) & 255
                plsc.addupdate_scatter(hnxt, [d2], jnp.ones((16,), jnp.int32))
        return c

    lax.fori_loop(0, nch // 8, perm_body, None)


def _sort_window(ha, hb, ka, pa, va, kb, pb, vb, kmin, kmax, nch):
    base = kmin & jnp.int32(-256)
    span = kmax - base
    _scan_hist(ha)
    _zero_hist(hb)
    _perm_pass(ha, hb, ka, pa, va, kb, pb, vb, 0, 8, base, nch)
    _scan_hist(hb)
    _zero_hist(ha)
    _perm_pass(hb, ha, kb, pb, vb, ka, pa, va, 8, 16, base, nch)

    @pl.when(span >= (1 << 16))
    def _():
        _scan_hist(ha)
        _zero_hist(hb)
        _perm_pass(ha, hb, ka, pa, va, kb, pb, vb, 16, 24, base, nch)

        @pl.when(span >= (1 << 24))
        def _():
            _scan_hist(hb)
            _perm_pass(hb, hb, kb, pb, vb, ka, pa, va, 24, None, base, nch)

    return (span >= (1 << 16)) & (span < (1 << 24))


def _minmax_hist16(ref, ha, nch):
    def body(j, carry):
        k = ref[pl.ds(j * 16, 16)]
        plsc.addupdate_scatter(ha, [k & 255], jnp.ones((16,), jnp.int32))
        return (jnp.minimum(carry[0], k), jnp.maximum(carry[1], k))

    init = (jnp.full((16,), jnp.int32(2**31 - 1)),
            jnp.full((16,), jnp.int32(-(2**31))))
    kmin_v, kmax_v = lax.fori_loop(0, nch, body, init)
    return jnp.min(kmin_v), jnp.max(kmax_v)


def _phase1_body(src_hbm, idx_hbm, mm_hbm, ko_hbm, po_hbm, vo_hbm,
                 mmv, seg, ka, pa, va, kb, pb, vb, ha, hb):
    wid = lax.axis_index("s") * 2 + lax.axis_index("c")
    lane = lax.iota(jnp.int32, 16)
    pltpu.sync_copy(mm_hbm, mmv)
    mn_vec = mmv[pl.ds(0, 16)]
    inv_vec = 1.0 / (mmv[pl.ds(128, 16)] - mn_vec + jnp.float32(EPS))

    def tile_body(i, _):
        w = i * NWORK + wid

        @pl.when(w < NW1)
        def _():
            base = w * B
            pltpu.sync_copy(src_hbm.at[pl.ds(base, B)], va)
            pltpu.sync_copy(idx_hbm.at[pl.ds(base, B)], seg)
            _zero_hist(ha)

            def key_body(j, carry):
                sl = pl.ds(j * 16, 16)
                nrm = (va[sl] - mn_vec) * inv_vec + seg[sl].astype(jnp.float32)
                kbits = plsc.bitcast(nrm, jnp.int32)
                ka[sl] = kbits
                pa[sl] = base + j * 16 + lane
                plsc.addupdate_scatter(ha, [kbits & 255],
                                       jnp.ones((16,), jnp.int32))
                return (jnp.minimum(carry[0], kbits), jnp.maximum(carry[1], kbits))

            init = (jnp.full((16,), jnp.int32(2**31 - 1)),
                    jnp.full((16,), jnp.int32(-(2**31))))
            kmin_v, kmax_v = lax.fori_loop(0, C, key_body, init)
            in_b = _sort_window(ha, hb, ka, pa, va, kb, pb, vb,
                                jnp.min(kmin_v), jnp.max(kmax_v), C)

            @pl.when(in_b)
            def _():
                pltpu.sync_copy(kb.at[pl.ds(0, FR)], ko_hbm.at[pl.ds(base, FR)])
                pltpu.sync_copy(kb.at[pl.ds(B - FR, FR)],
                                ko_hbm.at[pl.ds(base + B - FR, FR)])
                pltpu.sync_copy(pb, po_hbm.at[pl.ds(base, B)])
                pltpu.sync_copy(vb, vo_hbm.at[pl.ds(base, B)])

            @pl.when(jnp.logical_not(in_b))
            def _():
                pltpu.sync_copy(ka.at[pl.ds(0, FR)], ko_hbm.at[pl.ds(base, FR)])
                pltpu.sync_copy(ka.at[pl.ds(B - FR, FR)],
                                ko_hbm.at[pl.ds(base + B - FR, FR)])
                pltpu.sync_copy(pa, po_hbm.at[pl.ds(base, B)])
                pltpu.sync_copy(va, vo_hbm.at[pl.ds(base, B)])

        return _

    lax.fori_loop(0, WPT, tile_body, None)


def _phase2_body(ki_hbm, pi_hbm, vi_hbm, srt_hbm, perm_hbm,
                 ka, pa, va, kb, pb, vb, cp, cv, ha, hb):
    wid = lax.axis_index("s") * 2 + lax.axis_index("c")

    def copy_body(i, _):
        w = i * NWORK + wid

        @pl.when(w == 0)
        def _():
            pltpu.sync_copy(pi_hbm.at[pl.ds(0, B - FR)], cp.at[pl.ds(0, B - FR)])
            pltpu.sync_copy(vi_hbm.at[pl.ds(0, B - FR)], cv.at[pl.ds(0, B - FR)])
            pltpu.sync_copy(cp.at[pl.ds(0, B - FR)], perm_hbm.at[pl.ds(0, B - FR)])
            pltpu.sync_copy(cv.at[pl.ds(0, B - FR)], srt_hbm.at[pl.ds(0, B - FR)])

        @pl.when((w > 0) & (w < NW1 - 1))
        def _():
            s = w * B + FR
            pltpu.sync_copy(pi_hbm.at[pl.ds(s, B - FB)], cp.at[pl.ds(0, B - FB)])
            pltpu.sync_copy(vi_hbm.at[pl.ds(s, B - FB)], cv.at[pl.ds(0, B - FB)])
            pltpu.sync_copy(cp.at[pl.ds(0, B - FB)], perm_hbm.at[pl.ds(s, B - FB)])
            pltpu.sync_copy(cv.at[pl.ds(0, B - FB)], srt_hbm.at[pl.ds(s, B - FB)])

        @pl.when(w == NW1 - 1)
        def _():
            s = (NW1 - 1) * B + FR
            pltpu.sync_copy(pi_hbm.at[pl.ds(s, B - FR)], cp.at[pl.ds(0, B - FR)])
            pltpu.sync_copy(vi_hbm.at[pl.ds(s, B - FR)], cv.at[pl.ds(0, B - FR)])
            pltpu.sync_copy(cp.at[pl.ds(0, B - FR)], perm_hbm.at[pl.ds(s, B - FR)])
            pltpu.sync_copy(cv.at[pl.ds(0, B - FR)], srt_hbm.at[pl.ds(s, B - FR)])

        return _

    lax.fori_loop(0, WPT, copy_body, None)

    def fix_body(i, _):
        w = i * NWORK + wid

        @pl.when(w < NW2)
        def _():
            base = (w + 1) * B - FR
            pltpu.sync_copy(ki_hbm.at[pl.ds(base, FB)], ka)
            pltpu.sync_copy(pi_hbm.at[pl.ds(base, FB)], pa)
            pltpu.sync_copy(vi_hbm.at[pl.ds(base, FB)], va)
            _zero_hist(ha)
            kmin, kmax = _minmax_hist16(ka, ha, FC)
            in_b = _sort_window(ha, hb, ka, pa, va, kb, pb, vb, kmin, kmax, FC)

            @pl.when(in_b)
            def _():
                pltpu.sync_copy(pb, perm_hbm.at[pl.ds(base, FB)])
                pltpu.sync_copy(vb, srt_hbm.at[pl.ds(base, FB)])

            @pl.when(jnp.logical_not(in_b))
            def _():
                pltpu.sync_copy(pa, perm_hbm.at[pl.ds(base, FB)])
                pltpu.sync_copy(va, srt_hbm.at[pl.ds(base, FB)])

        return _

    lax.fori_loop(0, WPT, fix_body, None)


_phase1 = functools.partial(
    pl.kernel,
    out_type=[
        jax.ShapeDtypeStruct((N,), jnp.int32),
        jax.ShapeDtypeStruct((N,), jnp.int32),
        jax.ShapeDtypeStruct((N,), jnp.float32),
    ],
    mesh=_mesh,
    compiler_params=_params,
    scratch_types=[
        pltpu.VMEM((256,), jnp.float32),
        pltpu.VMEM((B,), jnp.int32),
        pltpu.VMEM((B,), jnp.int32),
        pltpu.VMEM((B,), jnp.int32),
        pltpu.VMEM((B,), jnp.float32),
        pltpu.VMEM((B,), jnp.int32),
        pltpu.VMEM((B,), jnp.int32),
        pltpu.VMEM((B,), jnp.float32),
        pltpu.VMEM((256,), jnp.int32),
        pltpu.VMEM((256,), jnp.int32),
    ],
)(_phase1_body)

_phase2 = functools.partial(
    pl.kernel,
    out_type=[
        jax.ShapeDtypeStruct((N,), jnp.float32),
        jax.ShapeDtypeStruct((N,), jnp.int32),
    ],
    mesh=_mesh,
    compiler_params=_params,
    scratch_types=[
        pltpu.VMEM((FB,), jnp.int32),
        pltpu.VMEM((FB,), jnp.int32),
        pltpu.VMEM((FB,), jnp.float32),
        pltpu.VMEM((FB,), jnp.int32),
        pltpu.VMEM((FB,), jnp.int32),
        pltpu.VMEM((FB,), jnp.float32),
        pltpu.VMEM((B,), jnp.int32),
        pltpu.VMEM((B,), jnp.float32),
        pltpu.VMEM((256,), jnp.int32),
        pltpu.VMEM((256,), jnp.int32),
    ],
)(_phase2_body)


def kernel(src, index):
    mm = pl.pallas_call(
        _mm_body,
        out_shape=jax.ShapeDtypeStruct((2, 128), jnp.float32),
    )(src.reshape(MMR, MMC))
    key1, pos1, val1 = _phase1(src, index.astype(jnp.int32), mm.reshape(256))
    srt, perm = _phase2(key1, pos1, val1)
    return srt, perm
